# baseline (speedup 1.0000x reference)
import jax
import jax.numpy as jnp
from jax import lax
from jax.experimental import pallas as pl
from jax.experimental.pallas import tpu as pltpu
from jax.experimental.pallas import tpu_sc as plsc

Q = 4096
D = 64
DP = 128
KS = 16
ROW = KS + 1
NTR = 1000000
NC, NS, L = 2, 16, 16
NW = NC * NS
QW = Q // NW
CQ = 8
NCH = QW // CQ
NSL = Q + 1
NSL_PAD = ((NSL + L - 1) // L) * L
CB = 8192


_EYE = None


def _tp_body(in_ref, eye_ref, out_ref):
    out_ref[:, 0:D] = jax.lax.dot_general(
        in_ref[...], eye_ref[...], (((0,), (0,)), ((), ())),
        precision=jax.lax.Precision.HIGHEST)


def _transpose_table(tt):
    grid = (NTR + CB - 1) // CB
    eye = jnp.eye(D, dtype=jnp.float32)
    return pl.pallas_call(
        _tp_body,
        grid=(grid,),
        in_specs=[pl.BlockSpec((D, CB), lambda i: (0, i)),
                  pl.BlockSpec((D, D), lambda i: (0, 0))],
        out_specs=pl.BlockSpec((CB, DP), lambda i: (i, 0)),
        out_shape=jax.ShapeDtypeStruct((NTR, DP), jnp.float32),
    )(tt, eye)


def _body(qf, ids, table, kvec, out, slices,
          qblk_v, idx_v, comb_v, slc_v, kv_v, sem):
    wid = lax.axis_index("s") * NC + lax.axis_index("c")
    q0w = wid * QW

    @pl.when(wid == 0)
    def _():
        pltpu.sync_copy(kvec, kv_v)
        step = kv_v[...] + 1

        def sbody(i, carry):
            off = pl.multiple_of(i * L, 8)
            slc_v[pl.ds(off, L)] = (lax.iota(jnp.int32, L) + i * L) * step
            return carry

        lax.fori_loop(0, NSL_PAD // L, sbody, 0)
        pltpu.sync_copy(slc_v.at[pl.ds(0, NSL)], slices)

    pltpu.sync_copy(qf.at[pl.ds(q0w, QW)], qblk_v)

    def chunk(c, carry):
        q0 = q0w + c * CQ
        i0 = pl.multiple_of(q0 * KS, 8)
        pltpu.sync_copy(ids.at[pl.ds(i0, CQ * KS)], idx_v)
        cps = []
        for j in range(CQ):
            idxj = idx_v[pl.ds(j * KS, KS)] * 2
            cps.append(pltpu.async_copy(
                table.at[idxj], comb_v.at[pl.ds(j * ROW + 1, KS)], sem))
        for j in range(CQ):
            r = c * CQ + j
            for t in range(D // L):
                comb_v[j * ROW, pl.ds(t * L, L)] = qblk_v[r, pl.ds(t * L, L)]
        for cp in cps:
            cp.wait()
        pltpu.sync_copy(comb_v, out.at[pl.ds(q0 * ROW, CQ * ROW)])
        return carry

    lax.fori_loop(0, NCH, chunk, 0)


@jax.jit
def _nn_gather(query_feats, ids_flat, train_table_t, kvec):
    table_rm = _transpose_table(train_table_t)
    table_lin = table_rm.reshape(2 * NTR, D)
    mesh = plsc.VectorSubcoreMesh(core_axis_name="c", subcore_axis_name="s")
    call = pl.kernel(
        _body,
        out_type=[
            jax.ShapeDtypeStruct((Q * ROW, D), jnp.float32),
            jax.ShapeDtypeStruct((NSL,), jnp.int32),
        ],
        mesh=mesh,
        scratch_types=[
            pltpu.VMEM((QW, D), jnp.float32),
            pltpu.VMEM((CQ * KS,), jnp.int32),
            pltpu.VMEM((CQ * ROW, D), jnp.float32),
            pltpu.VMEM((NSL_PAD,), jnp.int32),
            pltpu.VMEM((L,), jnp.int32),
            pltpu.SemaphoreType.DMA,
        ],
        compiler_params=pltpu.CompilerParams(use_tc_tiling_on_sc=False),
    )
    return call(query_feats, ids_flat, table_lin, kvec)


def kernel(query_feats, knn_ids, train_table, k):
    ids_flat = knn_ids.reshape(-1).astype(jnp.int32)
    kvec = jnp.full((L,), k, dtype=jnp.int32)
    neighbor_list, neighbor_slices = _nn_gather(
        query_feats, ids_flat, train_table.T, kvec)
    return neighbor_list, neighbor_slices


# speedup vs baseline: 1.6070x; 1.6070x over previous
import jax
import jax.numpy as jnp
from jax import lax
from jax.experimental import pallas as pl
from jax.experimental.pallas import tpu as pltpu
from jax.experimental.pallas import tpu_sc as plsc

Q = 4096
D = 64
DP = 128
KS = 16
ROW = KS + 1
NTR = 1000000
NC, NS, L = 2, 16, 16
NW = NC * NS
QW = Q // NW
CQ = 8
NCH = QW // CQ
NSL = Q + 1
NSL_PAD = ((NSL + L - 1) // L) * L
CB = 8192
SPL = 524288
NBLK = SPL // CB


def _tp_body(in_a, in_b, out_ref):
    out_ref[:, 0:D] = in_a[...].T
    out_ref[:, D:DP] = in_b[...].T


def _transpose_table(tt):
    return pl.pallas_call(
        _tp_body,
        grid=(NBLK,),
        in_specs=[pl.BlockSpec((D, CB), lambda i: (0, i)),
                  pl.BlockSpec(
                      (D, CB),
                      lambda i: (0, jnp.minimum(i + NBLK, (NTR - 1) // CB)))],
        out_specs=pl.BlockSpec((CB, DP), lambda i: (i, 0)),
        out_shape=jax.ShapeDtypeStruct((SPL, DP), jnp.float32),
    )(tt, tt)


def _body(qf, ids, table, kvec, out, slices,
          qblk_v, idx_v, comb_v, slc_v, kv_v, sem):
    wid = lax.axis_index("s") * NC + lax.axis_index("c")
    q0w = wid * QW

    @pl.when(wid == 0)
    def _():
        pltpu.sync_copy(kvec, kv_v)
        step = kv_v[...] + 1

        def sbody(i, carry):
            off = pl.multiple_of(i * L, 8)
            slc_v[pl.ds(off, L)] = (lax.iota(jnp.int32, L) + i * L) * step
            return carry

        lax.fori_loop(0, NSL_PAD // L, sbody, 0)
        pltpu.sync_copy(slc_v.at[pl.ds(0, NSL)], slices)

    pltpu.sync_copy(qf.at[pl.ds(q0w, QW)], qblk_v)

    def chunk(c, carry):
        q0 = q0w + c * CQ
        i0 = pl.multiple_of(q0 * KS, 8)
        pltpu.sync_copy(ids.at[pl.ds(i0, CQ * KS)], idx_v)
        cps = []
        for j in range(CQ):
            raw = idx_v[pl.ds(j * KS, KS)]
            idxj = raw * 2 - jnp.where(raw >= SPL, 2 * SPL - 1, 0)
            cps.append(pltpu.async_copy(
                table.at[idxj], comb_v.at[pl.ds(j * ROW + 1, KS)], sem))
        for j in range(CQ):
            r = c * CQ + j
            for t in range(D // L):
                comb_v[j * ROW, pl.ds(t * L, L)] = qblk_v[r, pl.ds(t * L, L)]
        for cp in cps:
            cp.wait()
        pltpu.sync_copy(comb_v, out.at[pl.ds(q0 * ROW, CQ * ROW)])
        return carry

    lax.fori_loop(0, NCH, chunk, 0)


@jax.jit
def _nn_gather(query_feats, ids_flat, train_table_t, kvec):
    table_rm = _transpose_table(train_table_t)
    table_lin = table_rm.reshape(2 * SPL, D)
    mesh = plsc.VectorSubcoreMesh(core_axis_name="c", subcore_axis_name="s")
    call = pl.kernel(
        _body,
        out_type=[
            jax.ShapeDtypeStruct((Q * ROW, D), jnp.float32),
            jax.ShapeDtypeStruct((NSL,), jnp.int32),
        ],
        mesh=mesh,
        scratch_types=[
            pltpu.VMEM((QW, D), jnp.float32),
            pltpu.VMEM((CQ * KS,), jnp.int32),
            pltpu.VMEM((CQ * ROW, D), jnp.float32),
            pltpu.VMEM((NSL_PAD,), jnp.int32),
            pltpu.VMEM((L,), jnp.int32),
            pltpu.SemaphoreType.DMA,
        ],
        compiler_params=pltpu.CompilerParams(use_tc_tiling_on_sc=False),
    )
    return call(query_feats, ids_flat, table_lin, kvec)


def kernel(query_feats, knn_ids, train_table, k):
    ids_flat = knn_ids.reshape(-1).astype(jnp.int32)
    kvec = jnp.full((L,), k, dtype=jnp.int32)
    neighbor_list, neighbor_slices = _nn_gather(
        query_feats, ids_flat, train_table.T, kvec)
    return neighbor_list, neighbor_slices


# speedup vs baseline: 1.6865x; 1.0494x over previous
import jax
import jax.numpy as jnp
from jax import lax
from jax.experimental import pallas as pl
from jax.experimental.pallas import tpu as pltpu
from jax.experimental.pallas import tpu_sc as plsc

Q = 4096
D = 64
DP = 128
KS = 16
ROW = KS + 1
NTR = 1000000
NC, NS, L = 2, 16, 16
NW = NC * NS
QW = Q // NW
CQ = 8
NCH = QW // CQ
NSL = Q + 1
NSL_PAD = ((NSL + L - 1) // L) * L
CB = 16384
SPL = 524288
NBLK = SPL // CB


def _tp_body(in_a, in_b, out_ref):
    out_ref[:, 0:D] = in_a[...].T
    out_ref[:, D:DP] = in_b[...].T


def _transpose_table(tt):
    return pl.pallas_call(
        _tp_body,
        grid=(NBLK,),
        in_specs=[pl.BlockSpec((D, CB), lambda i: (0, i)),
                  pl.BlockSpec(
                      (D, CB),
                      lambda i: (0, jnp.minimum(i + NBLK, (NTR - 1) // CB)))],
        out_specs=pl.BlockSpec((CB, DP), lambda i: (i, 0)),
        out_shape=jax.ShapeDtypeStruct((SPL, DP), jnp.float32),
    )(tt, tt)


def _body(qf, ids, table, kvec, out, slices,
          qblk_v, idx_v, comb_v, slc_v, kv_v, sem):
    wid = lax.axis_index("s") * NC + lax.axis_index("c")
    q0w = wid * QW

    @pl.when(wid == 0)
    def _():
        pltpu.sync_copy(kvec, kv_v)
        step = kv_v[...] + 1

        def sbody(i, carry):
            off = pl.multiple_of(i * L, 8)
            slc_v[pl.ds(off, L)] = (lax.iota(jnp.int32, L) + i * L) * step
            return carry

        lax.fori_loop(0, NSL_PAD // L, sbody, 0)
        pltpu.sync_copy(slc_v.at[pl.ds(0, NSL)], slices)

    pltpu.sync_copy(qf.at[pl.ds(q0w, QW)], qblk_v)

    def chunk(c, carry):
        q0 = q0w + c * CQ
        i0 = pl.multiple_of(q0 * KS, 8)
        pltpu.sync_copy(ids.at[pl.ds(i0, CQ * KS)], idx_v)
        cps = []
        for j in range(CQ):
            raw = idx_v[pl.ds(j * KS, KS)]
            idxj = raw * 2 - jnp.where(raw >= SPL, 2 * SPL - 1, 0)
            cps.append(pltpu.async_copy(
                table.at[idxj], comb_v.at[pl.ds(j * ROW + 1, KS)], sem))
        for j in range(CQ):
            r = c * CQ + j
            for t in range(D // L):
                comb_v[j * ROW, pl.ds(t * L, L)] = qblk_v[r, pl.ds(t * L, L)]
        for cp in cps:
            cp.wait()
        pltpu.sync_copy(comb_v, out.at[pl.ds(q0 * ROW, CQ * ROW)])
        return carry

    lax.fori_loop(0, NCH, chunk, 0)


@jax.jit
def _nn_gather(query_feats, ids_flat, train_table_t, kvec):
    table_rm = _transpose_table(train_table_t)
    table_lin = table_rm.reshape(2 * SPL, D)
    mesh = plsc.VectorSubcoreMesh(core_axis_name="c", subcore_axis_name="s")
    call = pl.kernel(
        _body,
        out_type=[
            jax.ShapeDtypeStruct((Q * ROW, D), jnp.float32),
            jax.ShapeDtypeStruct((NSL,), jnp.int32),
        ],
        mesh=mesh,
        scratch_types=[
            pltpu.VMEM((QW, D), jnp.float32),
            pltpu.VMEM((CQ * KS,), jnp.int32),
            pltpu.VMEM((CQ * ROW, D), jnp.float32),
            pltpu.VMEM((NSL_PAD,), jnp.int32),
            pltpu.VMEM((L,), jnp.int32),
            pltpu.SemaphoreType.DMA,
        ],
        compiler_params=pltpu.CompilerParams(use_tc_tiling_on_sc=False),
    )
    return call(query_feats, ids_flat, table_lin, kvec)


def kernel(query_feats, knn_ids, train_table, k):
    ids_flat = knn_ids.reshape(-1).astype(jnp.int32)
    kvec = jnp.full((L,), k, dtype=jnp.int32)
    neighbor_list, neighbor_slices = _nn_gather(
        query_feats, ids_flat, train_table.T, kvec)
    return neighbor_list, neighbor_slices


# speedup vs baseline: 1.7490x; 1.0371x over previous
import jax
import jax.numpy as jnp
from jax import lax
from jax.experimental import pallas as pl
from jax.experimental.pallas import tpu as pltpu
from jax.experimental.pallas import tpu_sc as plsc

Q = 4096
D = 64
DP = 128
KS = 16
ROW = KS + 1
NTR = 1000000
NC, NS, L = 2, 16, 16
NW = NC * NS
QW = Q // NW
CQ = 8
NCH = QW // CQ
NSL = Q + 1
NSL_PAD = ((NSL + L - 1) // L) * L
CB = 16384
SPL = 524288
NBLK = SPL // CB


def _tp_body(in_a, in_b, out_ref):
    out_ref[:, 0:D] = in_a[...].T
    out_ref[:, D:DP] = in_b[...].T


def _transpose_table(tt):
    return pl.pallas_call(
        _tp_body,
        grid=(NBLK,),
        in_specs=[pl.BlockSpec((D, CB), lambda i: (0, i)),
                  pl.BlockSpec(
                      (D, CB),
                      lambda i: (0, jnp.minimum(i + NBLK, (NTR - 1) // CB)))],
        out_specs=pl.BlockSpec((CB, DP), lambda i: (i, 0)),
        out_shape=jax.ShapeDtypeStruct((SPL, DP), jnp.float32),
    )(tt, tt)


def _body(qf, ids, table, kvec, out, slices,
          qblk_v, idx0_v, idx1_v, comb0_v, comb1_v, slc_v, kv_v,
          sem, sem_st0, sem_st1):
    wid = lax.axis_index("s") * NC + lax.axis_index("c")
    q0w = wid * QW
    idx_bufs = (idx0_v, idx1_v)
    comb_bufs = (comb0_v, comb1_v)
    st_sems = (sem_st0, sem_st1)

    @pl.when(wid == 0)
    def _():
        pltpu.sync_copy(kvec, kv_v)
        step = kv_v[...] + 1

        def sbody(i, carry):
            off = pl.multiple_of(i * L, 8)
            slc_v[pl.ds(off, L)] = (lax.iota(jnp.int32, L) + i * L) * step
            return carry

        lax.fori_loop(0, NSL_PAD // L, sbody, 0)
        pltpu.sync_copy(slc_v.at[pl.ds(0, NSL)], slices)

    pltpu.sync_copy(qf.at[pl.ds(q0w, QW)], qblk_v)
    pltpu.sync_copy(ids.at[pl.ds(pl.multiple_of(q0w * KS, 8), CQ * KS)],
                    idx0_v)

    def do_chunk(c, b, p):
        q0 = q0w + c * CQ
        idx_v, comb_v, sem_st = idx_bufs[b], comb_bufs[b], st_sems[b]
        @pl.when(p >= 1)
        def _():
            pltpu.make_async_copy(
                comb_v, out.at[pl.ds(0, CQ * ROW)], sem_st).wait()
        cps = []
        for j in range(CQ):
            raw = idx_v[pl.ds(j * KS, KS)]
            idxj = raw * 2 - jnp.where(raw >= SPL, 2 * SPL - 1, 0)
            cps.append(pltpu.async_copy(
                table.at[idxj], comb_v.at[pl.ds(j * ROW + 1, KS)], sem))
        @pl.when(c + 1 < NCH)
        def _():
            i1 = pl.multiple_of((q0 + CQ) * KS, 8)
            pltpu.sync_copy(ids.at[pl.ds(i1, CQ * KS)], idx_bufs[b ^ 1])
        for j in range(CQ):
            r = c * CQ + j
            for t in range(D // L):
                comb_v[j * ROW, pl.ds(t * L, L)] = qblk_v[r, pl.ds(t * L, L)]
        for cp in cps:
            cp.wait()
        pltpu.async_copy(comb_v, out.at[pl.ds(q0 * ROW, CQ * ROW)], sem_st)

    def pair(p, carry):
        do_chunk(2 * p, 0, p)
        do_chunk(2 * p + 1, 1, p)
        return carry

    lax.fori_loop(0, NCH // 2, pair, 0)
    pltpu.make_async_copy(comb0_v, out.at[pl.ds(0, CQ * ROW)], sem_st0).wait()
    pltpu.make_async_copy(comb1_v, out.at[pl.ds(0, CQ * ROW)], sem_st1).wait()


@jax.jit
def _nn_gather(query_feats, ids_flat, train_table_t, kvec):
    table_rm = _transpose_table(train_table_t)
    table_lin = table_rm.reshape(2 * SPL, D)
    mesh = plsc.VectorSubcoreMesh(core_axis_name="c", subcore_axis_name="s")
    call = pl.kernel(
        _body,
        out_type=[
            jax.ShapeDtypeStruct((Q * ROW, D), jnp.float32),
            jax.ShapeDtypeStruct((NSL,), jnp.int32),
        ],
        mesh=mesh,
        scratch_types=[
            pltpu.VMEM((QW, D), jnp.float32),
            pltpu.VMEM((CQ * KS,), jnp.int32),
            pltpu.VMEM((CQ * KS,), jnp.int32),
            pltpu.VMEM((CQ * ROW, D), jnp.float32),
            pltpu.VMEM((CQ * ROW, D), jnp.float32),
            pltpu.VMEM((NSL_PAD,), jnp.int32),
            pltpu.VMEM((L,), jnp.int32),
            pltpu.SemaphoreType.DMA,
            pltpu.SemaphoreType.DMA,
            pltpu.SemaphoreType.DMA,
        ],
        compiler_params=pltpu.CompilerParams(use_tc_tiling_on_sc=False),
    )
    return call(query_feats, ids_flat, table_lin, kvec)


def kernel(query_feats, knn_ids, train_table, k):
    ids_flat = knn_ids.reshape(-1).astype(jnp.int32)
    kvec = jnp.full((L,), k, dtype=jnp.int32)
    neighbor_list, neighbor_slices = _nn_gather(
        query_feats, ids_flat, train_table.T, kvec)
    return neighbor_list, neighbor_slices


# speedup vs baseline: 1.7649x; 1.0091x over previous
import jax
import jax.numpy as jnp
from jax import lax
from jax.experimental import pallas as pl
from jax.experimental.pallas import tpu as pltpu
from jax.experimental.pallas import tpu_sc as plsc

Q = 4096
D = 64
DP = 128
KS = 16
ROW = KS + 1
NTR = 1000000
NC, NS, L = 2, 16, 16
NW = NC * NS
QW = Q // NW
CQ = 16
NCH = QW // CQ
NSL = Q + 1
NSL_PAD = ((NSL + L - 1) // L) * L
CB = 16384
SPL = 524288
NBLK = SPL // CB


def _tp_body(in_a, in_b, out_ref):
    out_ref[:, 0:D] = in_a[...].T
    out_ref[:, D:DP] = in_b[...].T


def _transpose_table(tt):
    return pl.pallas_call(
        _tp_body,
        grid=(NBLK,),
        in_specs=[pl.BlockSpec((D, CB), lambda i: (0, i)),
                  pl.BlockSpec(
                      (D, CB),
                      lambda i: (0, jnp.minimum(i + NBLK, (NTR - 1) // CB)))],
        out_specs=pl.BlockSpec((CB, DP), lambda i: (i, 0)),
        out_shape=jax.ShapeDtypeStruct((SPL, DP), jnp.float32),
    )(tt, tt)


def _body(qf, ids, table, kvec, out, slices,
          qblk_v, idx0_v, idx1_v, comb0_v, comb1_v, slc_v, kv_v,
          sem, sem_st0, sem_st1):
    wid = lax.axis_index("s") * NC + lax.axis_index("c")
    q0w = wid * QW
    idx_bufs = (idx0_v, idx1_v)
    comb_bufs = (comb0_v, comb1_v)
    st_sems = (sem_st0, sem_st1)

    @pl.when(wid == 0)
    def _():
        pltpu.sync_copy(kvec, kv_v)
        step = kv_v[...] + 1

        def sbody(i, carry):
            off = pl.multiple_of(i * L, 8)
            slc_v[pl.ds(off, L)] = (lax.iota(jnp.int32, L) + i * L) * step
            return carry

        lax.fori_loop(0, NSL_PAD // L, sbody, 0)
        pltpu.sync_copy(slc_v.at[pl.ds(0, NSL)], slices)

    pltpu.sync_copy(qf.at[pl.ds(q0w, QW)], qblk_v)
    pltpu.sync_copy(ids.at[pl.ds(pl.multiple_of(q0w * KS, 8), CQ * KS)],
                    idx0_v)

    def do_chunk(c, b, p):
        q0 = q0w + c * CQ
        idx_v, comb_v, sem_st = idx_bufs[b], comb_bufs[b], st_sems[b]
        @pl.when(p >= 1)
        def _():
            pltpu.make_async_copy(
                comb_v, out.at[pl.ds(0, CQ * ROW)], sem_st).wait()
        cps = []
        for j in range(CQ):
            raw = idx_v[pl.ds(j * KS, KS)]
            idxj = raw * 2 - jnp.where(raw >= SPL, 2 * SPL - 1, 0)
            cps.append(pltpu.async_copy(
                table.at[idxj], comb_v.at[pl.ds(j * ROW + 1, KS)], sem))
        @pl.when(c + 1 < NCH)
        def _():
            i1 = pl.multiple_of((q0 + CQ) * KS, 8)
            pltpu.sync_copy(ids.at[pl.ds(i1, CQ * KS)], idx_bufs[b ^ 1])
        for j in range(CQ):
            r = c * CQ + j
            for t in range(D // L):
                comb_v[j * ROW, pl.ds(t * L, L)] = qblk_v[r, pl.ds(t * L, L)]
        for cp in cps:
            cp.wait()
        pltpu.async_copy(comb_v, out.at[pl.ds(q0 * ROW, CQ * ROW)], sem_st)

    def pair(p, carry):
        do_chunk(2 * p, 0, p)
        do_chunk(2 * p + 1, 1, p)
        return carry

    lax.fori_loop(0, NCH // 2, pair, 0)
    pltpu.make_async_copy(comb0_v, out.at[pl.ds(0, CQ * ROW)], sem_st0).wait()
    pltpu.make_async_copy(comb1_v, out.at[pl.ds(0, CQ * ROW)], sem_st1).wait()


@jax.jit
def _nn_gather(query_feats, ids_flat, train_table_t, kvec):
    table_rm = _transpose_table(train_table_t)
    table_lin = table_rm.reshape(2 * SPL, D)
    mesh = plsc.VectorSubcoreMesh(core_axis_name="c", subcore_axis_name="s")
    call = pl.kernel(
        _body,
        out_type=[
            jax.ShapeDtypeStruct((Q * ROW, D), jnp.float32),
            jax.ShapeDtypeStruct((NSL,), jnp.int32),
        ],
        mesh=mesh,
        scratch_types=[
            pltpu.VMEM((QW, D), jnp.float32),
            pltpu.VMEM((CQ * KS,), jnp.int32),
            pltpu.VMEM((CQ * KS,), jnp.int32),
            pltpu.VMEM((CQ * ROW, D), jnp.float32),
            pltpu.VMEM((CQ * ROW, D), jnp.float32),
            pltpu.VMEM((NSL_PAD,), jnp.int32),
            pltpu.VMEM((L,), jnp.int32),
            pltpu.SemaphoreType.DMA,
            pltpu.SemaphoreType.DMA,
            pltpu.SemaphoreType.DMA,
        ],
        compiler_params=pltpu.CompilerParams(use_tc_tiling_on_sc=False),
    )
    return call(query_feats, ids_flat, table_lin, kvec)


def kernel(query_feats, knn_ids, train_table, k):
    ids_flat = knn_ids.reshape(-1).astype(jnp.int32)
    kvec = jnp.full((L,), k, dtype=jnp.int32)
    neighbor_list, neighbor_slices = _nn_gather(
        query_feats, ids_flat, train_table.T, kvec)
    return neighbor_list, neighbor_slices


# speedup vs baseline: 1.7708x; 1.0033x over previous
import jax
import jax.numpy as jnp
from jax import lax
from jax.experimental import pallas as pl
from jax.experimental.pallas import tpu as pltpu
from jax.experimental.pallas import tpu_sc as plsc

Q = 4096
D = 64
DP = 128
KS = 16
ROW = KS + 1
NTR = 1000000
NC, NS, L = 2, 16, 16
NW = NC * NS
QW = Q // NW
CQ = 32
NCH = QW // CQ
NSL = Q + 1
NSL_PAD = ((NSL + L - 1) // L) * L
CB = 16384
SPL = 524288
NBLK = SPL // CB


def _tp_body(in_a, in_b, out_ref):
    out_ref[:, 0:D] = in_a[...].T
    out_ref[:, D:DP] = in_b[...].T


def _transpose_table(tt):
    return pl.pallas_call(
        _tp_body,
        grid=(NBLK,),
        in_specs=[pl.BlockSpec((D, CB), lambda i: (0, i)),
                  pl.BlockSpec(
                      (D, CB),
                      lambda i: (0, jnp.minimum(i + NBLK, (NTR - 1) // CB)))],
        out_specs=pl.BlockSpec((CB, DP), lambda i: (i, 0)),
        out_shape=jax.ShapeDtypeStruct((SPL, DP), jnp.float32),
    )(tt, tt)


def _body(qf, ids, table, kvec, out, slices,
          qblk_v, idx0_v, idx1_v, comb0_v, comb1_v, slc_v, kv_v,
          sem, sem_st0, sem_st1):
    wid = lax.axis_index("s") * NC + lax.axis_index("c")
    q0w = wid * QW
    idx_bufs = (idx0_v, idx1_v)
    comb_bufs = (comb0_v, comb1_v)
    st_sems = (sem_st0, sem_st1)

    @pl.when(wid == 0)
    def _():
        pltpu.sync_copy(kvec, kv_v)
        step = kv_v[...] + 1

        def sbody(i, carry):
            off = pl.multiple_of(i * L, 8)
            slc_v[pl.ds(off, L)] = (lax.iota(jnp.int32, L) + i * L) * step
            return carry

        lax.fori_loop(0, NSL_PAD // L, sbody, 0)
        pltpu.sync_copy(slc_v.at[pl.ds(0, NSL)], slices)

    pltpu.sync_copy(qf.at[pl.ds(q0w, QW)], qblk_v)
    pltpu.sync_copy(ids.at[pl.ds(pl.multiple_of(q0w * KS, 8), CQ * KS)],
                    idx0_v)

    def do_chunk(c, b, p):
        q0 = q0w + c * CQ
        idx_v, comb_v, sem_st = idx_bufs[b], comb_bufs[b], st_sems[b]
        @pl.when(p >= 1)
        def _():
            pltpu.make_async_copy(
                comb_v, out.at[pl.ds(0, CQ * ROW)], sem_st).wait()
        cps = []
        for j in range(CQ):
            raw = idx_v[pl.ds(j * KS, KS)]
            idxj = raw * 2 - jnp.where(raw >= SPL, 2 * SPL - 1, 0)
            cps.append(pltpu.async_copy(
                table.at[idxj], comb_v.at[pl.ds(j * ROW + 1, KS)], sem))
        @pl.when(c + 1 < NCH)
        def _():
            i1 = pl.multiple_of((q0 + CQ) * KS, 8)
            pltpu.sync_copy(ids.at[pl.ds(i1, CQ * KS)], idx_bufs[b ^ 1])
        for j in range(CQ):
            r = c * CQ + j
            for t in range(D // L):
                comb_v[j * ROW, pl.ds(t * L, L)] = qblk_v[r, pl.ds(t * L, L)]
        for cp in cps:
            cp.wait()
        pltpu.async_copy(comb_v, out.at[pl.ds(q0 * ROW, CQ * ROW)], sem_st)

    def pair(p, carry):
        do_chunk(2 * p, 0, p)
        do_chunk(2 * p + 1, 1, p)
        return carry

    lax.fori_loop(0, NCH // 2, pair, 0)
    pltpu.make_async_copy(comb0_v, out.at[pl.ds(0, CQ * ROW)], sem_st0).wait()
    pltpu.make_async_copy(comb1_v, out.at[pl.ds(0, CQ * ROW)], sem_st1).wait()


@jax.jit
def _nn_gather(query_feats, ids_flat, train_table_t, kvec):
    table_rm = _transpose_table(train_table_t)
    table_lin = table_rm.reshape(2 * SPL, D)
    mesh = plsc.VectorSubcoreMesh(core_axis_name="c", subcore_axis_name="s")
    call = pl.kernel(
        _body,
        out_type=[
            jax.ShapeDtypeStruct((Q * ROW, D), jnp.float32),
            jax.ShapeDtypeStruct((NSL,), jnp.int32),
        ],
        mesh=mesh,
        scratch_types=[
            pltpu.VMEM((QW, D), jnp.float32),
            pltpu.VMEM((CQ * KS,), jnp.int32),
            pltpu.VMEM((CQ * KS,), jnp.int32),
            pltpu.VMEM((CQ * ROW, D), jnp.float32),
            pltpu.VMEM((CQ * ROW, D), jnp.float32),
            pltpu.VMEM((NSL_PAD,), jnp.int32),
            pltpu.VMEM((L,), jnp.int32),
            pltpu.SemaphoreType.DMA,
            pltpu.SemaphoreType.DMA,
            pltpu.SemaphoreType.DMA,
        ],
        compiler_params=pltpu.CompilerParams(use_tc_tiling_on_sc=False),
    )
    return call(query_feats, ids_flat, table_lin, kvec)


def kernel(query_feats, knn_ids, train_table, k):
    ids_flat = knn_ids.reshape(-1).astype(jnp.int32)
    kvec = jnp.full((L,), k, dtype=jnp.int32)
    neighbor_list, neighbor_slices = _nn_gather(
        query_feats, ids_flat, train_table.T, kvec)
    return neighbor_list, neighbor_slices
